# baseline (speedup 1.0000x reference)
import jax
import jax.numpy as jnp
from jax import lax
from jax.experimental import pallas as pl
from jax.experimental.pallas import tpu as pltpu
from jax.experimental.pallas import tpu_sc as plsc

NUM_ENTITIES = 1000000
EMBED_DIM = 64
BATCH = 16384

NC, NS, L = 2, 16, 16
NW = NC * NS
B_PER_W = BATCH // NW
CHUNK = 64
N_CHUNKS = B_PER_W // CHUNK
FUSED = 2 * EMBED_DIM


def _body(h_hbm, r_hbm, t_hbm, er_hbm, ei_hbm, rr_hbm, ri_hbm, out_hbm,
          hj0, hh0, tj0, th0, rj0, rh0,
          hj1, hh1, tj1, th1, rj1, rh1,
          g00, g01, g02, g03, g04, g05,
          g10, g11, g12, g13, g14, g15,
          score_b, ibounce, sem0, sem1):
    idx = [(hj0, hh0, tj0, th0, rj0, rh0), (hj1, hh1, tj1, th1, rj1, rh1)]
    bufs = [(g00, g01, g02, g03, g04, g05), (g10, g11, g12, g13, g14, g15)]
    sems = [sem0, sem1]

    wid = lax.axis_index("s") * NC + lax.axis_index("c")
    base = wid * B_PER_W
    lanes = lax.iota(jnp.int32, L)

    def prep(hbm, cbase, j_ref, half_ref):
        pltpu.sync_copy(hbm.at[pl.ds(cbase, CHUNK)], ibounce)
        for g in range(CHUNK // L):
            s = pl.ds(g * L, L)
            v = ibounce[s]
            j_ref[s] = lax.shift_right_logical(v, 1)
            half_ref[s] = lax.shift_left(v & 1, 6)

    def issue(c, k):
        cbase = base + c * CHUNK
        hj, hh, tj, th, rj, rh = idx[k]
        prep(h_hbm, cbase, hj, hh)
        prep(t_hbm, cbase, tj, th)
        prep(r_hbm, cbase, rj, rh)
        bhr, bhi, btr, bti, brr, bri = bufs[k]
        sem = sems[k]
        pltpu.async_copy(er_hbm.at[hj], bhr, sem)
        pltpu.async_copy(ei_hbm.at[hj], bhi, sem)
        pltpu.async_copy(er_hbm.at[tj], btr, sem)
        pltpu.async_copy(ei_hbm.at[tj], bti, sem)
        pltpu.async_copy(rr_hbm.at[rj], brr, sem)
        pltpu.async_copy(ri_hbm.at[rj], bri, sem)

    def drain(k):
        sem = sems[k]
        src = er_hbm.at[pl.ds(0, CHUNK)]
        for buf in bufs[k]:
            pltpu.make_async_copy(src, buf, sem).wait()

    def compute(c, k):
        hj, hh, tj, th, rj, rh = idx[k]
        bhr, bhi, btr, bti, brr, bri = bufs[k]
        for g in range(CHUNK // L):
            s = pl.ds(g * L, L)
            rows = g * L + lanes
            vhh = hh[s]
            vth = th[s]
            vrh = rh[s]

            def dstep(d, acc):
                ch = vhh + d
                ct = vth + d
                cr = vrh + d
                vhr = plsc.load_gather(bhr, [rows, ch])
                vhi = plsc.load_gather(bhi, [rows, ch])
                vtr = plsc.load_gather(btr, [rows, ct])
                vti = plsc.load_gather(bti, [rows, ct])
                vrr = plsc.load_gather(brr, [rows, cr])
                vri = plsc.load_gather(bri, [rows, cr])
                p1 = vhr * vtr + vhi * vti
                p2 = vhi * vtr - vhr * vti
                return acc + ((1.0 + vrr) * p1 - vri * p2)

            acc = lax.fori_loop(0, EMBED_DIM, dstep,
                                jnp.zeros((L,), jnp.float32), unroll=4)
            score_b[pl.ds(c * CHUNK + g * L, L)] = acc

    issue(0, 0)
    for c in range(N_CHUNKS):
        k = c % 2
        drain(k)
        if c + 1 < N_CHUNKS:
            issue(c + 1, 1 - k)
        compute(c, k)

    pltpu.sync_copy(score_b, out_hbm.at[pl.ds(base, B_PER_W)])


@jax.jit
def _complex_score(h, r, t, er2, ei2, rr2, ri2):
    mesh = plsc.VectorSubcoreMesh(core_axis_name="c", subcore_axis_name="s")
    ibuf = pltpu.VMEM((CHUNK,), jnp.int32)
    gbuf = pltpu.VMEM((CHUNK, FUSED), jnp.float32)
    kern = pl.kernel(
        _body,
        out_type=jax.ShapeDtypeStruct((BATCH,), jnp.float32),
        mesh=mesh,
        compiler_params=pltpu.CompilerParams(needs_layout_passes=False),
        scratch_types=[
            ibuf, ibuf, ibuf, ibuf, ibuf, ibuf,
            ibuf, ibuf, ibuf, ibuf, ibuf, ibuf,
            gbuf, gbuf, gbuf, gbuf, gbuf, gbuf,
            gbuf, gbuf, gbuf, gbuf, gbuf, gbuf,
            pltpu.VMEM((B_PER_W,), jnp.float32),
            ibuf,
            pltpu.SemaphoreType.DMA,
            pltpu.SemaphoreType.DMA,
        ],
    )
    return kern(h, r, t, er2, ei2, rr2, ri2)


def kernel(h, r, t, ent_real, ent_imag, rel_real, rel_imag):
    h = h.astype(jnp.int32)
    r = r.astype(jnp.int32)
    t = t.astype(jnp.int32)
    er2 = ent_real.reshape(NUM_ENTITIES // 2, FUSED)
    ei2 = ent_imag.reshape(NUM_ENTITIES // 2, FUSED)
    rr2 = rel_real.reshape(-1, FUSED)
    ri2 = rel_imag.reshape(-1, FUSED)
    return _complex_score(h, r, t, er2, ei2, rr2, ri2)


# speedup vs baseline: 1.4103x; 1.4103x over previous
import jax
import jax.numpy as jnp
from jax import lax
from jax.experimental import pallas as pl
from jax.experimental.pallas import tpu as pltpu
from jax.experimental.pallas import tpu_sc as plsc

NUM_ENTITIES = 1000000
EMBED_DIM = 64
BATCH = 16384

NC, NS, L = 2, 16, 16
NW = NC * NS
B_PER_W = BATCH // NW
CHUNK = 8
N_CHUNKS = B_PER_W // CHUNK
TILE = 8


def _body(h_hbm, r_hbm, t_hbm, er_hbm, ei_hbm, rr_hbm, ri_hbm, out_hbm,
          h_s0, r_s0, t_s0, h_s1, r_s1, t_s1,
          g00, g01, g02, g03, g04, g05,
          g10, g11, g12, g13, g14, g15,
          psum_b, score_b, ibounce, sem0, sem1):
    idx = [(h_s0, r_s0, t_s0), (h_s1, r_s1, t_s1)]
    bufs = [(g00, g01, g02, g03, g04, g05), (g10, g11, g12, g13, g14, g15)]
    sems = [sem0, sem1]

    wid = lax.axis_index("s") * NC + lax.axis_index("c")
    base = wid * B_PER_W
    lanes = lax.iota(jnp.int32, L)

    def to_smem(hbm, cbase, sm_ref):
        pltpu.sync_copy(hbm.at[pl.ds(cbase, CHUNK)], ibounce.at[pl.ds(0, CHUNK)])
        v = ibounce[pl.ds(0, L)]
        for j in range(CHUNK):
            sm_ref[j] = jnp.sum(jnp.where(lanes == j, v, 0))

    def issue(cbase, k):
        h_s, r_s, t_s = idx[k]
        to_smem(h_hbm, cbase, h_s)
        to_smem(r_hbm, cbase, r_s)
        to_smem(t_hbm, cbase, t_s)
        hr_b, hi_b, tr_b, ti_b, rr_b, ri_b = bufs[k]
        sem = sems[k]
        for i in range(CHUNK):
            hb = (h_s[i] // TILE) * TILE
            rb = (r_s[i] // TILE) * TILE
            tb = (t_s[i] // TILE) * TILE
            d = pl.ds(i * TILE, TILE)
            pltpu.async_copy(er_hbm.at[pl.ds(hb, TILE)], hr_b.at[d], sem)
            pltpu.async_copy(ei_hbm.at[pl.ds(hb, TILE)], hi_b.at[d], sem)
            pltpu.async_copy(er_hbm.at[pl.ds(tb, TILE)], tr_b.at[d], sem)
            pltpu.async_copy(ei_hbm.at[pl.ds(tb, TILE)], ti_b.at[d], sem)
            pltpu.async_copy(rr_hbm.at[pl.ds(rb, TILE)], rr_b.at[d], sem)
            pltpu.async_copy(ri_hbm.at[pl.ds(rb, TILE)], ri_b.at[d], sem)

    def drain(k):
        sem = sems[k]
        src = rr_hbm.at[pl.ds(0, CHUNK * TILE)]
        for buf in bufs[k]:
            pltpu.make_async_copy(src, buf, sem).wait()

    def compute(cidx, k):
        h_s, r_s, t_s = idx[k]
        hr_b, hi_b, tr_b, ti_b, rr_b, ri_b = bufs[k]
        for i in range(CHUNK):
            hm = lax.rem(h_s[i], TILE) + i * TILE
            rm = lax.rem(r_s[i], TILE) + i * TILE
            tm = lax.rem(t_s[i], TILE) + i * TILE
            acc = None
            for j in range(EMBED_DIM // L):
                s = pl.ds(j * L, L)
                vhr = hr_b[hm, s]
                vhi = hi_b[hm, s]
                vtr = tr_b[tm, s]
                vti = ti_b[tm, s]
                vrr = rr_b[rm, s]
                vri = ri_b[rm, s]
                p1 = vhr * vtr + vhi * vti
                p2 = vhi * vtr - vhr * vti
                term = (1.0 + vrr) * p1 - vri * p2
                acc = term if acc is None else acc + term
            psum_b[pl.ds((cidx * CHUNK + i) * L, L)] = acc

    issue(base, 0)
    issue(base + CHUNK, 1)

    def step(it, carry):
        c0 = 2 * it
        drain(0)
        compute(c0, 0)

        @pl.when(c0 + 2 < N_CHUNKS)
        def _():
            issue(base + (c0 + 2) * CHUNK, 0)

        drain(1)
        compute(c0 + 1, 1)

        @pl.when(c0 + 3 < N_CHUNKS)
        def _():
            issue(base + (c0 + 3) * CHUNK, 1)

        return carry

    lax.fori_loop(0, N_CHUNKS // 2, step, 0)

    for g in range(B_PER_W // L):
        rows = g * L + lanes
        acc = None
        for d in range(L):
            fidx = rows * L + d
            v = plsc.load_gather(psum_b, [fidx])
            acc = v if acc is None else acc + v
        score_b[pl.ds(g * L, L)] = acc

    pltpu.sync_copy(score_b, out_hbm.at[pl.ds(base, B_PER_W)])


@jax.jit
def _complex_score(h, r, t, ent_real, ent_imag, rel_real, rel_imag):
    mesh = plsc.VectorSubcoreMesh(core_axis_name="c", subcore_axis_name="s")
    ibuf = pltpu.SMEM((CHUNK,), jnp.int32)
    gbuf = pltpu.VMEM((CHUNK * TILE, EMBED_DIM), jnp.float32)
    kern = pl.kernel(
        _body,
        out_type=jax.ShapeDtypeStruct((BATCH,), jnp.float32),
        mesh=mesh,
        compiler_params=pltpu.CompilerParams(needs_layout_passes=False),
        scratch_types=[
            ibuf, ibuf, ibuf, ibuf, ibuf, ibuf,
            gbuf, gbuf, gbuf, gbuf, gbuf, gbuf,
            gbuf, gbuf, gbuf, gbuf, gbuf, gbuf,
            pltpu.VMEM((B_PER_W * L,), jnp.float32),
            pltpu.VMEM((B_PER_W,), jnp.float32),
            pltpu.VMEM((L,), jnp.int32),
            pltpu.SemaphoreType.DMA,
            pltpu.SemaphoreType.DMA,
        ],
    )
    return kern(h, r, t, ent_real, ent_imag, rel_real, rel_imag)


def kernel(h, r, t, ent_real, ent_imag, rel_real, rel_imag):
    h = h.astype(jnp.int32)
    r = r.astype(jnp.int32)
    t = t.astype(jnp.int32)
    return _complex_score(h, r, t, ent_real, ent_imag, rel_real, rel_imag)
